# baseline (speedup 1.0000x reference)
import functools
import math

import jax
import jax.numpy as jnp
from jax import lax
from jax.experimental import pallas as pl
from jax.experimental.pallas import tpu as pltpu
from jax.experimental.pallas import tpu_sc as plsc

_VOCAB = 1000
_VOCAB_PAD = 1024
_EMB = 128
_BATCH = 4096
_SEQ = 200
_SCALE = float(_EMB) ** 0.5

_NC = 2
_NS = 16
_NW = _NC * _NS

_ROWS = _BATCH * _SEQ
_ROWS_PER_W = _ROWS // _NW
_CHUNK = _SEQ
_NCHUNK = _ROWS_PER_W // _CHUNK
_SUB = 40
_NSUB = _CHUNK // _SUB
_TROWS = _VOCAB_PAD // _NS
_NBUF = 4


def _sinusoid_signal():
    position = jnp.arange(_SEQ, dtype=jnp.float32)
    num_ts = _EMB // 2
    inc = math.log(10000.0) / (num_ts - 1)
    inv_ts = jnp.exp(jnp.arange(num_ts, dtype=jnp.float32) * -inc)
    scaled = position[:, None] * inv_ts[None, :]
    return jnp.concatenate([jnp.sin(scaled), jnp.cos(scaled)], axis=1)


@functools.partial(
    pl.kernel,
    out_type=jax.ShapeDtypeStruct((_ROWS, _EMB), jnp.float32),
    mesh=plsc.VectorSubcoreMesh(core_axis_name="c", subcore_axis_name="s"),
    scratch_types=(
        [pltpu.VMEM((_CHUNK,), jnp.int32)] * _NBUF
        + [pltpu.VMEM((_CHUNK,), jnp.int32)]
        + [pltpu.VMEM((_CHUNK, _EMB), jnp.float32)] * _NBUF
        + [
            pltpu.VMEM_SHARED((_VOCAB_PAD, _EMB), jnp.float32),
        ]
        + [pltpu.SemaphoreType.DMA] * (4 * _NBUF)
    ),
)
def _embed_kernel(idx_hbm, table_hbm, sig_hbm, sig_idx_hbm, out_hbm, *refs):
    idx_v = refs[0:_NBUF]
    sig_idx_v = refs[_NBUF]
    rows_v = refs[_NBUF + 1:2 * _NBUF + 1]
    table_sp = refs[2 * _NBUF + 1]
    sems = refs[2 * _NBUF + 2:]
    sem_g = sems[0:_NBUF]
    sem_o = sems[_NBUF:2 * _NBUF]
    sem_i = sems[2 * _NBUF:3 * _NBUF]
    sem_a = sems[3 * _NBUF:4 * _NBUF]

    sid = lax.axis_index("s")
    wid = sid * _NC + lax.axis_index("c")
    row_base_w = wid * _ROWS_PER_W

    trow = sid * _TROWS
    pltpu.sync_copy(table_hbm.at[pl.ds(trow, _TROWS)],
                    rows_v[0].at[pl.ds(0, _TROWS)])

    def scale_body(r, c2):
        for c in range(_EMB // 16):
            sl = pl.ds(c * 16, 16)
            rows_v[0][r, sl] = rows_v[0][r, sl] * _SCALE
        return c2

    lax.fori_loop(0, _TROWS, scale_body, 0, unroll=False)
    pltpu.sync_copy(rows_v[0].at[pl.ds(0, _TROWS)],
                    table_sp.at[pl.ds(trow, _TROWS)])

    pltpu.sync_copy(sig_idx_hbm, sig_idx_v)
    plsc.subcore_barrier()

    def start_idx(q, b):
        row_base = row_base_w + q * _CHUNK
        pltpu.async_copy(idx_hbm.at[pl.ds(row_base, _CHUNK)], idx_v[b],
                         sem_i[b])

    def wait_idx(b):
        pltpu.make_async_copy(
            idx_hbm.at[pl.ds(0, _CHUNK)], idx_v[b], sem_i[b]
        ).wait()

    def start_gather(b):
        for j in range(_NSUB):
            pltpu.async_copy(
                table_sp.at[idx_v[b].at[pl.ds(j * _SUB, _SUB)]],
                rows_v[b].at[pl.ds(j * _SUB, _SUB)],
                sem_g[b],
            )

    def wait_gather(b):
        pltpu.make_async_copy(
            table_hbm.at[pl.ds(0, _CHUNK)], rows_v[b], sem_g[b]
        ).wait()

    def start_sig_add(b):
        for j in range(_NSUB):
            pltpu.async_copy(
                sig_hbm.at[sig_idx_v.at[pl.ds(j * _SUB, _SUB)]],
                rows_v[b].at[pl.ds(j * _SUB, _SUB)],
                sem_a[b],
                add=True,
            )

    def wait_sig_add(b):
        pltpu.make_async_copy(
            table_hbm.at[pl.ds(0, _CHUNK)], rows_v[b], sem_a[b]
        ).wait()

    def wait_out(b):
        pltpu.make_async_copy(
            rows_v[b], out_hbm.at[pl.ds(0, _CHUNK)], sem_o[b]
        ).wait()

    pltpu.sync_copy(idx_hbm.at[pl.ds(row_base_w, _CHUNK)], idx_v[0])
    start_gather(0)
    for k in (1, 2):
        start_idx(k, k)

    def ring_body(g, carry):
        for b in range(_NBUF):
            q = _NBUF * g + b
            b1 = (b + 1) % _NBUF
            b3 = (b + 3) % _NBUF

            wait_gather(b)
            start_sig_add(b)

            @pl.when(q + 1 < _NCHUNK)
            def _prefetch_gather():
                wait_idx(b1)

                @pl.when(q >= _NBUF - 1)
                def _():
                    wait_out(b1)
                start_gather(b1)

            @pl.when(q + 3 < _NCHUNK)
            def _prefetch_idx():
                start_idx(q + 3, b3)

            wait_sig_add(b)
            row_base = row_base_w + q * _CHUNK
            pltpu.async_copy(rows_v[b], out_hbm.at[pl.ds(row_base, _CHUNK)],
                             sem_o[b])
        return carry

    lax.fori_loop(0, _NCHUNK // _NBUF, ring_body, 0, unroll=False)
    for k in (1, 2, 3):
        wait_out((_NCHUNK - _NBUF + k) % _NBUF)


def kernel(inputs, table):
    idx = inputs.astype(jnp.int32).reshape(_ROWS)
    table_p = jnp.pad(table, ((0, _VOCAB_PAD - _VOCAB), (0, 0)))
    sig = _sinusoid_signal()
    sig_idx = jnp.arange(_SEQ, dtype=jnp.int32)
    out = _embed_kernel(idx, table_p, sig, sig_idx)
    return out.reshape(_BATCH, _SEQ, _EMB)


# speedup vs baseline: 1.0703x; 1.0703x over previous
import functools
import math

import jax
import jax.numpy as jnp
from jax import lax
from jax.experimental import pallas as pl
from jax.experimental.pallas import tpu as pltpu
from jax.experimental.pallas import tpu_sc as plsc

_VOCAB = 1000
_VOCAB_PAD = 1024
_EMB = 128
_BATCH = 4096
_SEQ = 200
_SCALE = float(_EMB) ** 0.5

_NC = 2
_NS = 16
_NW = _NC * _NS

_ROWS = _BATCH * _SEQ
_ROWS_PER_W = _ROWS // _NW
_CHUNK = 160
_NCHUNK = _ROWS_PER_W // _CHUNK
_SUB = 40
_NSUB = _CHUNK // _SUB
_TROWS = _VOCAB_PAD // _NS
_NBUF = 4


def _sinusoid_signal():
    position = jnp.arange(_SEQ, dtype=jnp.float32)
    num_ts = _EMB // 2
    inc = math.log(10000.0) / (num_ts - 1)
    inv_ts = jnp.exp(jnp.arange(num_ts, dtype=jnp.float32) * -inc)
    scaled = position[:, None] * inv_ts[None, :]
    return jnp.concatenate([jnp.sin(scaled), jnp.cos(scaled)], axis=1)


@functools.partial(
    pl.kernel,
    out_type=jax.ShapeDtypeStruct((_ROWS, _EMB), jnp.float32),
    mesh=plsc.VectorSubcoreMesh(core_axis_name="c", subcore_axis_name="s"),
    scratch_types=(
        [pltpu.VMEM((_CHUNK,), jnp.int32)] * _NBUF
        + [pltpu.VMEM((_CHUNK, _EMB), jnp.float32)] * _NBUF
        + [
            pltpu.VMEM((_SEQ, _EMB), jnp.float32),
            pltpu.VMEM_SHARED((_VOCAB_PAD, _EMB), jnp.float32),
        ]
        + [pltpu.SemaphoreType.DMA] * (3 * _NBUF)
    ),
)
def _embed_kernel(idx_hbm, table_hbm, sig_hbm, out_hbm, *refs):
    idx_v = refs[0:_NBUF]
    rows_v = refs[_NBUF:2 * _NBUF]
    sig_v = refs[2 * _NBUF]
    table_sp = refs[2 * _NBUF + 1]
    sems = refs[2 * _NBUF + 2:]
    sem_g = sems[0:_NBUF]
    sem_o = sems[_NBUF:2 * _NBUF]
    sem_i = sems[2 * _NBUF:3 * _NBUF]

    sid = lax.axis_index("s")
    wid = sid * _NC + lax.axis_index("c")
    row_base_w = wid * _ROWS_PER_W

    trow = sid * _TROWS
    pltpu.sync_copy(table_hbm.at[pl.ds(trow, _TROWS)],
                    rows_v[0].at[pl.ds(0, _TROWS)])

    def scale_body(r, c2):
        for c in range(_EMB // 16):
            sl = pl.ds(c * 16, 16)
            rows_v[0][r, sl] = rows_v[0][r, sl] * _SCALE
        return c2

    lax.fori_loop(0, _TROWS, scale_body, 0, unroll=False)
    pltpu.sync_copy(rows_v[0].at[pl.ds(0, _TROWS)],
                    table_sp.at[pl.ds(trow, _TROWS)])
    pltpu.sync_copy(sig_hbm, sig_v)
    plsc.subcore_barrier()

    def start_idx(q, b):
        row_base = row_base_w + q * _CHUNK
        pltpu.async_copy(idx_hbm.at[pl.ds(row_base, _CHUNK)], idx_v[b],
                         sem_i[b])

    def wait_idx(b):
        pltpu.make_async_copy(
            idx_hbm.at[pl.ds(0, _CHUNK)], idx_v[b], sem_i[b]
        ).wait()

    def prefill(q, b):
        p = lax.rem(q * _CHUNK, _SEQ)
        len1 = lax.min(jnp.int32(_CHUNK), _SEQ - p)

        def seg1(r, c2):
            for c in range(_EMB // 16):
                sl = pl.ds(c * 16, 16)
                rows_v[b][r, sl] = sig_v[p + r, sl]
            return c2

        def seg2(r, c2):
            for c in range(_EMB // 16):
                sl = pl.ds(c * 16, 16)
                rows_v[b][r, sl] = sig_v[p + r - _SEQ, sl]
            return c2

        lax.fori_loop(0, len1, seg1, 0)
        lax.fori_loop(len1, _CHUNK, seg2, 0)

    def start_gadd(b):
        for j in range(_NSUB):
            pltpu.async_copy(
                table_sp.at[idx_v[b].at[pl.ds(j * _SUB, _SUB)]],
                rows_v[b].at[pl.ds(j * _SUB, _SUB)],
                sem_g[b],
                add=True,
            )

    def wait_gadd(b):
        pltpu.make_async_copy(
            table_hbm.at[pl.ds(0, _CHUNK)], rows_v[b], sem_g[b]
        ).wait()

    def wait_out(b):
        pltpu.make_async_copy(
            rows_v[b], out_hbm.at[pl.ds(0, _CHUNK)], sem_o[b]
        ).wait()

    pltpu.sync_copy(idx_hbm.at[pl.ds(row_base_w, _CHUNK)], idx_v[0])
    prefill(0, 0)
    start_gadd(0)
    for k in (1, 2):
        start_idx(k, k)

    def ring_body(g, carry):
        for b in range(_NBUF):
            q = _NBUF * g + b
            b1 = (b + 1) % _NBUF
            b3 = (b + 3) % _NBUF

            @pl.when(q + 1 < _NCHUNK)
            def _prep_next():
                @pl.when(q >= _NBUF - 1)
                def _():
                    wait_out(b1)
                prefill(q + 1, b1)
                wait_idx(b1)
                start_gadd(b1)

            @pl.when(q + 3 < _NCHUNK)
            def _prefetch_idx():
                start_idx(q + 3, b3)

            wait_gadd(b)
            row_base = row_base_w + q * _CHUNK
            pltpu.async_copy(rows_v[b], out_hbm.at[pl.ds(row_base, _CHUNK)],
                             sem_o[b])
        return carry

    lax.fori_loop(0, _NCHUNK // _NBUF, ring_body, 0, unroll=False)
    for k in (1, 2, 3):
        wait_out((_NCHUNK - _NBUF + k) % _NBUF)


def kernel(inputs, table):
    idx = inputs.astype(jnp.int32).reshape(_ROWS)
    table_p = jnp.pad(table, ((0, _VOCAB_PAD - _VOCAB), (0, 0)))
    sig = _sinusoid_signal()
    out = _embed_kernel(idx, table_p, sig)
    return out.reshape(_BATCH, _SEQ, _EMB)


# speedup vs baseline: 1.2199x; 1.1397x over previous
import functools
import math

import jax
import jax.numpy as jnp
from jax import lax
from jax.experimental import pallas as pl
from jax.experimental.pallas import tpu as pltpu
from jax.experimental.pallas import tpu_sc as plsc

_VOCAB = 1000
_VOCAB_PAD = 1024
_EMB = 128
_BATCH = 4096
_SEQ = 200
_SCALE = float(_EMB) ** 0.5

_NC = 2
_NS = 16
_NW = _NC * _NS

_ROWS = _BATCH * _SEQ
_ROWS_PER_W = _ROWS // _NW
_CHUNK = 160
_NCHUNK = _ROWS_PER_W // _CHUNK
_SUB = 40
_NSUB = _CHUNK // _SUB
_TROWS = _VOCAB_PAD // _NS
_NBUF = 4


def _sinusoid_signal():
    position = jnp.arange(_SEQ, dtype=jnp.float32)
    num_ts = _EMB // 2
    inc = math.log(10000.0) / (num_ts - 1)
    inv_ts = jnp.exp(jnp.arange(num_ts, dtype=jnp.float32) * -inc)
    scaled = position[:, None] * inv_ts[None, :]
    return jnp.concatenate([jnp.sin(scaled), jnp.cos(scaled)], axis=1)


@functools.partial(
    pl.kernel,
    out_type=jax.ShapeDtypeStruct((_ROWS, _EMB), jnp.float32),
    mesh=plsc.VectorSubcoreMesh(core_axis_name="c", subcore_axis_name="s"),
    scratch_types=(
        [pltpu.VMEM((_CHUNK,), jnp.int32)] * _NBUF
        + [pltpu.VMEM((_CHUNK, _EMB), jnp.float32)] * _NBUF
        + [
            pltpu.VMEM((_SEQ, _EMB), jnp.float32),
            pltpu.VMEM_SHARED((_VOCAB_PAD, _EMB), jnp.float32),
        ]
        + [pltpu.SemaphoreType.DMA] * (3 * _NBUF)
    ),
)
def _embed_kernel(idx_hbm, table_hbm, sig_hbm, out_hbm, *refs):
    idx_v = refs[0:_NBUF]
    rows_v = refs[_NBUF:2 * _NBUF]
    sig_v = refs[2 * _NBUF]
    table_sp = refs[2 * _NBUF + 1]
    sems = refs[2 * _NBUF + 2:]
    sem_g = sems[0:_NBUF]
    sem_o = sems[_NBUF:2 * _NBUF]
    sem_i = sems[2 * _NBUF:3 * _NBUF]

    sid = lax.axis_index("s")
    wid = sid * _NC + lax.axis_index("c")
    row_base_w = wid * _ROWS_PER_W

    trow = sid * _TROWS
    pltpu.sync_copy(table_hbm.at[pl.ds(trow, _TROWS)],
                    rows_v[0].at[pl.ds(0, _TROWS)])

    def scale_body(r, c2):
        for c in range(_EMB // 16):
            sl = pl.ds(c * 16, 16)
            rows_v[0][r, sl] = rows_v[0][r, sl] * _SCALE
        return c2

    lax.fori_loop(0, _TROWS, scale_body, 0, unroll=False)
    pltpu.sync_copy(rows_v[0].at[pl.ds(0, _TROWS)],
                    table_sp.at[pl.ds(trow, _TROWS)])
    pltpu.sync_copy(sig_hbm, sig_v)
    plsc.subcore_barrier()

    def start_idx(q, b):
        row_base = row_base_w + q * _CHUNK
        pltpu.async_copy(idx_hbm.at[pl.ds(row_base, _CHUNK)], idx_v[b],
                         sem_i[b])

    def wait_idx(b):
        pltpu.make_async_copy(
            idx_hbm.at[pl.ds(0, _CHUNK)], idx_v[b], sem_i[b]
        ).wait()

    def prefill(p, b):
        len1 = min(_CHUNK, _SEQ - p)

        def seg1(r, c2):
            for c in range(_EMB // 16):
                sl = pl.ds(c * 16, 16)
                rows_v[b][r, sl] = sig_v[p + r, sl]
            return c2

        lax.fori_loop(0, len1, seg1, 0, unroll=4)
        if len1 < _CHUNK:
            def seg2(r, c2):
                for c in range(_EMB // 16):
                    sl = pl.ds(c * 16, 16)
                    rows_v[b][len1 + r, sl] = sig_v[r, sl]
                return c2

            lax.fori_loop(0, _CHUNK - len1, seg2, 0, unroll=4)

    def start_gadd(b):
        for j in range(_NSUB):
            pltpu.async_copy(
                table_sp.at[idx_v[b].at[pl.ds(j * _SUB, _SUB)]],
                rows_v[b].at[pl.ds(j * _SUB, _SUB)],
                sem_g[b],
                add=True,
            )

    def wait_gadd(b):
        pltpu.make_async_copy(
            table_hbm.at[pl.ds(0, _CHUNK)], rows_v[b], sem_g[b]
        ).wait()

    def wait_out(b):
        pltpu.make_async_copy(
            rows_v[b], out_hbm.at[pl.ds(0, _CHUNK)], sem_o[b]
        ).wait()

    _GROUP = 20
    pltpu.sync_copy(idx_hbm.at[pl.ds(row_base_w, _CHUNK)], idx_v[0])
    prefill(0, 0)
    start_gadd(0)
    for k in (1, 2):
        start_idx(k, k)

    def ring_body(g, carry):
        for k in range(_GROUP):
            q = _GROUP * g + k
            b = k % _NBUF
            b1 = (b + 1) % _NBUF
            b3 = (b + 3) % _NBUF
            p1 = (_CHUNK * (k + 1)) % _SEQ

            @pl.when(q + 1 < _NCHUNK)
            def _prep_next():
                @pl.when(q >= _NBUF - 1)
                def _():
                    wait_out(b1)
                prefill(p1, b1)
                wait_idx(b1)
                start_gadd(b1)

            @pl.when(q + 3 < _NCHUNK)
            def _prefetch_idx():
                start_idx(q + 3, b3)

            wait_gadd(b)
            row_base = row_base_w + q * _CHUNK
            pltpu.async_copy(rows_v[b], out_hbm.at[pl.ds(row_base, _CHUNK)],
                             sem_o[b])
        return carry

    lax.fori_loop(0, _NCHUNK // _GROUP, ring_body, 0, unroll=False)
    for k in (1, 2, 3):
        wait_out((_NCHUNK - _NBUF + k) % _NBUF)


def kernel(inputs, table):
    idx = inputs.astype(jnp.int32).reshape(_ROWS)
    table_p = jnp.pad(table, ((0, _VOCAB_PAD - _VOCAB), (0, 0)))
    sig = _sinusoid_signal()
    out = _embed_kernel(idx, table_p, sig)
    return out.reshape(_BATCH, _SEQ, _EMB)


# speedup vs baseline: 2.9852x; 2.4471x over previous
import functools
import math

import jax
import jax.numpy as jnp
from jax import lax
from jax.experimental import pallas as pl
from jax.experimental.pallas import tpu as pltpu
from jax.experimental.pallas import tpu_sc as plsc

_VOCAB = 1000
_VOCAB_PAD = 1024
_EMB = 128
_BATCH = 4096
_SEQ = 200
_SCALE = float(_EMB) ** 0.5

_NC = 2
_NS = 16
_NW = _NC * _NS

_ROWS = _BATCH * _SEQ
_ROWS_PER_W = _ROWS // _NW
_CHUNK = _SEQ
_NCHUNK = _ROWS_PER_W // _CHUNK
_SUB = 40
_NSUB = _CHUNK // _SUB
_TROWS = _VOCAB_PAD // _NS
_NBUF = 3


def _sinusoid_signal():
    position = jnp.arange(_SEQ, dtype=jnp.float32)
    num_ts = _EMB // 2
    inc = math.log(10000.0) / (num_ts - 1)
    inv_ts = jnp.exp(jnp.arange(num_ts, dtype=jnp.float32) * -inc)
    scaled = position[:, None] * inv_ts[None, :]
    return jnp.concatenate([jnp.sin(scaled), jnp.cos(scaled)], axis=1)


@functools.partial(
    pl.kernel,
    out_type=jax.ShapeDtypeStruct((_ROWS, _EMB), jnp.float32),
    mesh=plsc.VectorSubcoreMesh(core_axis_name="c", subcore_axis_name="s"),
    scratch_types=(
        [pltpu.VMEM((_CHUNK,), jnp.int32)] * _NBUF
        + [pltpu.VMEM((_CHUNK, _EMB), jnp.float32)] * _NBUF
        + [
            pltpu.VMEM((_SEQ, _EMB), jnp.float32),
            pltpu.VMEM_SHARED((_VOCAB_PAD, _EMB), jnp.float32),
        ]
        + [pltpu.SemaphoreType.DMA] * (3 * _NBUF)
    ),
)
def _embed_kernel(idx_hbm, table_hbm, sig_hbm, out_hbm, *refs):
    idx_v = refs[0:_NBUF]
    rows_v = refs[_NBUF:2 * _NBUF]
    sig_v = refs[2 * _NBUF]
    table_sp = refs[2 * _NBUF + 1]
    sems = refs[2 * _NBUF + 2:]
    sem_g = sems[0:_NBUF]
    sem_o = sems[_NBUF:2 * _NBUF]
    sem_i = sems[2 * _NBUF:3 * _NBUF]

    sid = lax.axis_index("s")
    wid = sid * _NC + lax.axis_index("c")
    row_base_w = wid * _ROWS_PER_W

    trow = sid * _TROWS
    pltpu.sync_copy(table_hbm.at[pl.ds(trow, _TROWS)],
                    rows_v[0].at[pl.ds(0, _TROWS)])

    def scale_body(r, c2):
        for c in range(_EMB // 16):
            sl = pl.ds(c * 16, 16)
            rows_v[0][r, sl] = rows_v[0][r, sl] * _SCALE
        return c2

    lax.fori_loop(0, _TROWS, scale_body, 0, unroll=False)
    pltpu.sync_copy(rows_v[0].at[pl.ds(0, _TROWS)],
                    table_sp.at[pl.ds(trow, _TROWS)])
    pltpu.sync_copy(sig_hbm, sig_v)
    plsc.subcore_barrier()

    def start_idx(q, b):
        row_base = row_base_w + q * _CHUNK
        pltpu.async_copy(idx_hbm.at[pl.ds(row_base, _CHUNK)], idx_v[b],
                         sem_i[b])

    def wait_idx(b):
        pltpu.make_async_copy(
            idx_hbm.at[pl.ds(0, _CHUNK)], idx_v[b], sem_i[b]
        ).wait()

    def start_gather(b):
        for j in range(_NSUB):
            pltpu.async_copy(
                table_sp.at[idx_v[b].at[pl.ds(j * _SUB, _SUB)]],
                rows_v[b].at[pl.ds(j * _SUB, _SUB)],
                sem_g[b],
            )

    def wait_gather(b):
        pltpu.make_async_copy(
            table_hbm.at[pl.ds(0, _CHUNK)], rows_v[b], sem_g[b]
        ).wait()

    def wait_out(b):
        pltpu.make_async_copy(
            rows_v[b], out_hbm.at[pl.ds(0, _CHUNK)], sem_o[b]
        ).wait()

    def add_signal(b):
        def row_body(s, c2):
            for c in range(_EMB // 16):
                sl = pl.ds(c * 16, 16)
                plsc.addupdate(rows_v[b].at[s, sl], sig_v[s, sl])
            return c2

        lax.fori_loop(0, _CHUNK, row_body, 0, unroll=4)

    def chunk_iter(q, b):
        b1 = (b + 1) % _NBUF
        b2 = (b + 2) % _NBUF

        @pl.when(q + 1 < _NCHUNK)
        def _prefetch_gather():
            wait_idx(b1)

            @pl.when(q >= _NBUF - 1)
            def _():
                wait_out(b1)
            start_gather(b1)

        @pl.when(q + 2 < _NCHUNK)
        def _prefetch_idx():
            start_idx(q + 2, b2)

        wait_gather(b)
        add_signal(b)
        row_base = row_base_w + q * _CHUNK
        pltpu.async_copy(rows_v[b], out_hbm.at[pl.ds(row_base, _CHUNK)],
                         sem_o[b])

    pltpu.sync_copy(idx_hbm.at[pl.ds(row_base_w, _CHUNK)], idx_v[0])
    start_gather(0)
    start_idx(1, 1)

    def ring_body(g, carry):
        for b in range(_NBUF):
            chunk_iter(_NBUF * g + b, b)
        return carry

    _NFULL = (_NCHUNK // _NBUF) * _NBUF
    lax.fori_loop(0, _NCHUNK // _NBUF, ring_body, 0, unroll=False)
    for q in range(_NFULL, _NCHUNK):
        chunk_iter(q, q % _NBUF)
    for q in range(_NCHUNK - 2, _NCHUNK):
        wait_out(q % _NBUF)


def kernel(inputs, table):
    idx = inputs.astype(jnp.int32).reshape(_ROWS)
    table_p = jnp.pad(table, ((0, _VOCAB_PAD - _VOCAB), (0, 0)))
    sig = _sinusoid_signal()
    out = _embed_kernel(idx, table_p, sig)
    return out.reshape(_BATCH, _SEQ, _EMB)
